# baseline (speedup 1.0000x reference)
import functools

import jax
import jax.numpy as jnp
import numpy as np
from jax import lax
from jax.experimental import pallas as pl
from jax.experimental.pallas import tpu as pltpu
from jax.experimental.pallas import tpu_sc as plsc

_C, _H, _W = 9, 480, 640
_NV = 2 * _C * _H * _W
_NVP = _NV + 8
_DUMP = _NV
_N = 2_000_000
_CH = 2048
_NW = 32
_NFULL = _N // _CH
_TAIL_EV = _N - _NFULL * _CH
_TAIL_V = _TAIL_EV // 16
_PLANE = float(_W * _H * _C)
_BINSZ = float(_W * _H)
_BOUNDS = [np.float32(i / 9.0) for i in range(1, 9)]

_mesh = plsc.VectorSubcoreMesh(core_axis_name="c", subcore_axis_name="s")


@functools.partial(
    pl.kernel,
    out_type=(),
    mesh=_mesh,
    scratch_types=[
        pltpu.VMEM((_CH,), jnp.float32),
        pltpu.VMEM((_CH,), jnp.float32),
        pltpu.VMEM((_CH,), jnp.float32),
        pltpu.VMEM((_CH,), jnp.float32),
        pltpu.VMEM((16384,), jnp.int32),
        pltpu.VMEM((16384,), jnp.float32),
        pltpu.SemaphoreType.DMA,
    ],
)
def _voxelize(evs, out, xbuf, ybuf, tbuf, pbuf, idxbuf, ones, sem):
    wid = lax.axis_index("s") * 2 + lax.axis_index("c")
    one_v = jnp.full((16,), 1.0, dtype=jnp.float32)
    for g in range(16384 // 16):
        ones[pl.ds(g * 16, 16)] = one_v

    def compute_group(v):
        s = pl.ds(v * 16, 16)
        x = xbuf[s]
        y = ybuf[s]
        t = tbuf[s]
        p = pbuf[s]
        f = x + jnp.float32(_W) * y
        f = f + jnp.where(p > 0.0, jnp.float32(_PLANE), jnp.float32(0.0))
        for b in _BOUNDS:
            f = f + jnp.where(t > b, jnp.float32(_BINSZ), jnp.float32(0.0))
        idx = f.astype(jnp.int32)
        safe = jnp.where(t > 0.0, idx, jnp.int32(_DUMP))
        idxbuf[s] = safe

    iota16 = jnp.arange(16, dtype=jnp.int32)
    for g in range(16384 // 16):
        idxbuf[pl.ds(g * 16, 16)] = iota16 + g * 16 + wid * 16384

    def do_chunk(cid, ngroups):
        del cid, ngroups
        pltpu.async_copy(ones, out.at[idxbuf], sem).wait()

    def outer(c, _):
        do_chunk(wid + c * _NW, _CH // 16)
        return None

    lax.fori_loop(0, 4, outer, None)


def kernel(events):
    evs = events.T
    vox_ref = jax.new_ref(jnp.zeros((_NVP,), jnp.float32))
    _voxelize(evs, vox_ref)
    vox = vox_ref[...]
    return vox[:_NV].reshape(1, 2, _C, _H, _W)


# speedup vs baseline: 2.6209x; 2.6209x over previous
import functools

import jax
import jax.numpy as jnp
import numpy as np
from jax import lax
from jax.experimental import pallas as pl
from jax.experimental.pallas import tpu as pltpu
from jax.experimental.pallas import tpu_sc as plsc

_C, _H, _W = 9, 480, 640
_NV = 2 * _C * _H * _W
_NR = 4
_RSZ = _NV // _NR
_PAD = 8
_ZS = _RSZ // 16
_ZB = _ZS // 8
_N = 2_000_000
_CH = 2048
_NFULL = _N // _CH
_TAIL_EV = _N - _NFULL * _CH
_TAIL_V = _TAIL_EV // 16
_PLANE = float(_W * _H * _C)
_BINSZ = float(_W * _H)
_BOUNDS = [np.float32(i / 9.0) for i in range(1, 9)]

_mesh = plsc.VectorSubcoreMesh(core_axis_name="c", subcore_axis_name="s")


@functools.partial(
    pl.kernel,
    out_type=jax.ShapeDtypeStruct((_NV,), jnp.float32),
    mesh=_mesh,
    scratch_types=[
        pltpu.VMEM((_CH,), jnp.float32),
        pltpu.VMEM((_CH,), jnp.float32),
        pltpu.VMEM((_CH,), jnp.float32),
        pltpu.VMEM((_CH,), jnp.float32),
        pltpu.VMEM((_CH,), jnp.int32),
        pltpu.VMEM((_CH,), jnp.float32),
        pltpu.VMEM((_ZB,), jnp.float32),
        pltpu.VMEM_SHARED((_RSZ + _PAD,), jnp.float32),
        pltpu.SemaphoreType.DMA,
    ],
)
def _voxelize(evs, out, xbuf, ybuf, tbuf, pbuf, idxbuf, ones, zbuf, acc, sem):
    core = lax.axis_index("c")
    tile = lax.axis_index("s")
    one_v = jnp.full((16,), 1.0, dtype=jnp.float32)
    zero_v = jnp.zeros((16,), dtype=jnp.float32)
    for g in range(_CH // 16):
        ones[pl.ds(g * 16, 16)] = one_v
    for g in range(_ZB // 16):
        zbuf[pl.ds(g * 16, 16)] = zero_v

    def compute_group(v, rbase):
        s = pl.ds(v * 16, 16)
        x = xbuf[s]
        y = ybuf[s]
        t = tbuf[s]
        p = pbuf[s]
        f = x + jnp.float32(_W) * y
        f = f + jnp.where(p > 0.0, jnp.float32(_PLANE), jnp.float32(0.0))
        for b in _BOUNDS:
            f = f + jnp.where(t > b, jnp.float32(_BINSZ), jnp.float32(0.0))
        rel = f.astype(jnp.int32) - rbase
        ok = (t > 0.0) & (rel >= 0) & (rel < _RSZ)
        idxbuf[s] = jnp.where(ok, rel, jnp.int32(_RSZ))

    def do_chunk(cid, ngroups, rbase):
        nev = ngroups * 16
        base = cid * _CH
        for fld, buf in ((0, xbuf), (1, ybuf), (2, tbuf), (3, pbuf)):
            pltpu.sync_copy(evs.at[fld, pl.ds(base, nev)],
                            buf.at[pl.ds(0, nev)])
        lax.fori_loop(0, ngroups,
                      lambda v, _: (compute_group(v, rbase), None)[1], None)
        pltpu.sync_copy(ones, acc.at[idxbuf])

    for r in range(2):
        rbase = (core * 2 + r) * _RSZ
        for k in range(8):
            pltpu.sync_copy(zbuf, acc.at[pl.ds(tile * _ZS + k * _ZB, _ZB)])
        @pl.when(tile == 0)
        def _():
            pltpu.sync_copy(zbuf.at[pl.ds(0, _PAD)],
                            acc.at[pl.ds(_RSZ, _PAD)])
        plsc.subcore_barrier()
        def outer(k, _):
            do_chunk(tile + k * 16, _CH // 16, rbase)
            return None
        lax.fori_loop(0, _NFULL // 16, outer, None)
        @pl.when(tile == 0)
        def _():
            do_chunk(jnp.int32(_NFULL), _TAIL_V, rbase)
        plsc.subcore_barrier()
        pltpu.sync_copy(
            acc.at[pl.ds(tile * _ZS, _ZS)],
            out.at[pl.ds((core * 2 + r) * _RSZ + tile * _ZS, _ZS)])


def kernel(events):
    evs = events.T
    vox = _voxelize(evs)
    return vox.reshape(1, 2, _C, _H, _W)
